# baseline (speedup 1.0000x reference)
import functools

import jax
import jax.numpy as jnp
from jax import lax
from jax.experimental import pallas as pl
from jax.experimental.pallas import tpu as pltpu
from jax.experimental.pallas import tpu_sc as plsc

_N = 1_000_000
_NW = 32
_CHUNK = 31264
_ITERS = _CHUNK // 16

_LOGC = (7.0376836292e-2, -1.1514610310e-1, 1.1676998740e-1,
         -1.2420140846e-1, 1.4249322787e-1, -1.6668057665e-1,
         2.0000714765e-1, -2.4999993993e-1, 3.3333331174e-1)


def _ln(x):
    bits = lax.bitcast_convert_type(x, jnp.int32)
    e = (bits >> 23) - 126
    m = lax.bitcast_convert_type((bits & 0x007FFFFF) | 0x3F000000, jnp.float32)
    small = m < jnp.float32(0.7071067811865476)
    e = e - jnp.where(small, 1, 0)
    f = jnp.where(small, m + m, m) - jnp.float32(1.0)
    z = f * f
    poly = jnp.full((16,), _LOGC[0], jnp.float32)
    for c in _LOGC[1:]:
        poly = poly * f + jnp.float32(c)
    ef = e.astype(jnp.float32)
    y = f * z * poly
    y = y + jnp.float32(-2.12194440e-4) * ef
    y = y - jnp.float32(0.5) * z
    return f + y + jnp.float32(0.693359375) * ef


_mesh = plsc.VectorSubcoreMesh(core_axis_name="c", subcore_axis_name="s")


@functools.partial(
    pl.kernel,
    mesh=_mesh,
    out_type=(jax.ShapeDtypeStruct((_NW, 16), jnp.float32),
              jax.ShapeDtypeStruct((_NW, 16), jnp.int32)),
    scratch_types=(pltpu.VMEM((_CHUNK,), jnp.float32),
                   pltpu.VMEM((_CHUNK,), jnp.float32),
                   pltpu.VMEM((16,), jnp.float32),
                   pltpu.VMEM((16,), jnp.int32)),
)
def _sc_scan(p_hbm, u_hbm, vals_out, idx_out, p_v, u_v, rv, ri):
    w = lax.axis_index("s") * 2 + lax.axis_index("c")
    b = (w * (_N // _NW)) & -16
    b = pl.multiple_of(b, 16)
    pltpu.sync_copy(p_hbm.at[pl.ds(b, _CHUNK)], p_v)
    pltpu.sync_copy(u_hbm.at[pl.ds(b, _CHUNK)], u_v)
    lane = lax.iota(jnp.int32, 16)

    def body(i, carry):
        bv, bi = carry
        off = i * 16
        pv = p_v[pl.ds(off, 16)]
        uv = u_v[pl.ds(off, 16)]
        v = (jnp.float32(0.0) - _ln(uv)) * jnp.exp(-pv)
        idx = lane + (b + off)
        lt = v < bv
        return jnp.where(lt, v, bv), jnp.where(lt, idx, bi)

    init = (jnp.full((16,), jnp.inf, jnp.float32), jnp.zeros((16,), jnp.int32))
    bv, bi = lax.fori_loop(0, _ITERS, body, init)
    rv[...] = bv
    ri[...] = bi
    pltpu.sync_copy(rv, vals_out.at[w])
    pltpu.sync_copy(ri, idx_out.at[w])


def _merge_body(v_ref, i_ref, o_ref):
    v = v_ref[...]
    ix = i_ref[...]
    m = jnp.min(v)
    cand = jnp.where(v == m, ix, jnp.int32(2**31 - 1))
    o_ref[0, 0] = jnp.min(cand)


_merge = pl.pallas_call(
    _merge_body,
    out_shape=jax.ShapeDtypeStruct((1, 1), jnp.int32),
    out_specs=pl.BlockSpec(memory_space=pltpu.SMEM),
)


def kernel(p, u):
    vals, idx = _sc_scan(p, u)
    out = _merge(vals.reshape(4, 128), idx.reshape(4, 128))
    return out[0, 0]


# speedup vs baseline: 1.2226x; 1.2226x over previous
import functools

import jax
import jax.numpy as jnp
from jax import lax
from jax.experimental import pallas as pl
from jax.experimental.pallas import tpu as pltpu
from jax.experimental.pallas import tpu_sc as plsc

_N = 1_000_000
_NW = 32
_UNROLL = 4
_CHUNK = 31296
_ITERS = _CHUNK // (16 * _UNROLL)
_STRIDE = _N // _NW

_OFF = 0x3F3504F3
_LN2 = 0.6931471805599453
_PC = (0.1193119419053133, -0.18680964217965043, 0.2049179463920517,
       -0.24908270227751894, 0.33314670851721606, -0.5000114538020157,
       1.000000964626097)


def _neg_ln(x):
    bits = lax.bitcast_convert_type(x, jnp.int32)
    e = (bits - _OFF) >> 23
    m = lax.bitcast_convert_type(bits - (e << 23), jnp.float32)
    f = m - jnp.float32(1.0)
    poly = jnp.full((16,), _PC[0], jnp.float32)
    for c in _PC[1:]:
        poly = poly * f + jnp.float32(c)
    return jnp.float32(0.0) - (f * poly + e.astype(jnp.float32) * jnp.float32(_LN2))


_mesh = plsc.VectorSubcoreMesh(core_axis_name="c", subcore_axis_name="s")


@functools.partial(
    pl.kernel,
    mesh=_mesh,
    out_type=(jax.ShapeDtypeStruct((_NW, 16), jnp.float32),
              jax.ShapeDtypeStruct((_NW, 16), jnp.int32)),
    scratch_types=(pltpu.VMEM((_CHUNK,), jnp.float32),
                   pltpu.VMEM((_CHUNK,), jnp.float32),
                   pltpu.VMEM((16,), jnp.float32),
                   pltpu.VMEM((16,), jnp.int32),
                   pltpu.SemaphoreType.DMA,
                   pltpu.SemaphoreType.DMA),
)
def _sc_scan(p_hbm, u_hbm, vals_out, idx_out, p_v, u_v, rv, ri, sem_p, sem_u):
    w = lax.axis_index("s") * 2 + lax.axis_index("c")
    b = jnp.minimum((w * _STRIDE) & -16, _N - _CHUNK)
    b = pl.multiple_of(b, 16)
    cp_p = pltpu.async_copy(p_hbm.at[pl.ds(b, _CHUNK)], p_v, sem_p)
    cp_u = pltpu.async_copy(u_hbm.at[pl.ds(b, _CHUNK)], u_v, sem_u)
    cp_p.wait()
    cp_u.wait()

    def body(i, carry):
        bv, bc = carry
        base_code = i * _UNROLL
        v = []
        for j in range(_UNROLL):
            off = base_code + j
            pv = p_v[pl.ds(off * 16, 16)]
            uv = u_v[pl.ds(off * 16, 16)]
            v.append(_neg_ln(uv) * jnp.exp(jnp.float32(0.0) - pv))
        lt01 = v[1] < v[0]
        va = jnp.where(lt01, v[1], v[0])
        ca = jnp.where(lt01, base_code + 1, base_code)
        lt23 = v[3] < v[2]
        vb = jnp.where(lt23, v[3], v[2])
        cb = jnp.where(lt23, base_code + 3, base_code + 2)
        ltab = vb < va
        vw = jnp.where(ltab, vb, va)
        cw = jnp.where(ltab, cb, ca)
        lt = vw < bv
        return jnp.where(lt, vw, bv), jnp.where(lt, cw, bc)

    init = (jnp.full((16,), jnp.inf, jnp.float32), jnp.zeros((16,), jnp.int32))
    bv, bc = lax.fori_loop(0, _ITERS, body, init)
    rv[...] = bv
    ri[...] = b + bc * 16 + lax.iota(jnp.int32, 16)
    pltpu.sync_copy(rv, vals_out.at[w])
    pltpu.sync_copy(ri, idx_out.at[w])


def _merge_body(v_ref, i_ref, o_ref):
    v = v_ref[...]
    ix = i_ref[...]
    m = jnp.min(v)
    cand = jnp.where(v == m, ix, jnp.int32(2**31 - 1))
    o_ref[0, 0] = jnp.min(cand)


_merge = pl.pallas_call(
    _merge_body,
    out_shape=jax.ShapeDtypeStruct((1, 1), jnp.int32),
    out_specs=pl.BlockSpec(memory_space=pltpu.SMEM),
)


def kernel(p, u):
    vals, idx = _sc_scan(p, u)
    out = _merge(vals.reshape(4, 128), idx.reshape(4, 128))
    return out[0, 0]
